# baseline (speedup 1.0000x reference)
import functools

import jax
import jax.numpy as jnp
from jax.experimental import pallas as pl

B = 1024
N_CHANS = 64
WINDOW = 1000
N_OUT = 4
E = 16
FEAT = 512

BB = 64


def _tc_body(x_ref, sid_ref, Wbb_ref, bbb_ref, Wall_ref, ball_ref, out_ref):
    xb = x_ref[...]
    m = jnp.sum(xb, axis=2) * (1.0 / WINDOW)
    feats = jnp.dot(m, Wbb_ref[...], preferred_element_type=jnp.float32)
    feats = jnp.maximum(feats + bbb_ref[...], 0.0)
    allh = jnp.dot(feats, Wall_ref[...], preferred_element_type=jnp.float32)
    allh = allh + ball_ref[...]
    sid = sid_ref[...]
    lane = jax.lax.broadcasted_iota(jnp.int32, (BB, E * N_OUT), 1)
    mask = (lane // N_OUT == sid).astype(jnp.float32)
    masked = allh * mask
    jo = jax.lax.broadcasted_iota(jnp.int32, (E * N_OUT, N_OUT), 0)
    oo = jax.lax.broadcasted_iota(jnp.int32, (E * N_OUT, N_OUT), 1)
    sel = (jo % N_OUT == oo).astype(jnp.float32)
    out_ref[...] = jnp.dot(masked, sel, preferred_element_type=jnp.float32)


@jax.jit
def kernel(x, subject_ids, W_bb, b_bb, W_heads, b_heads):
    sid2 = subject_ids.astype(jnp.int32).reshape(B, 1)
    W_all = W_heads.transpose(1, 0, 2).reshape(FEAT, E * N_OUT)
    b_all = b_heads.reshape(1, E * N_OUT)
    bbb = b_bb.reshape(1, FEAT)
    grid = (B // BB,)
    return pl.pallas_call(
        _tc_body,
        grid=grid,
        in_specs=[
            pl.BlockSpec((BB, N_CHANS, WINDOW), lambda i: (i, 0, 0)),
            pl.BlockSpec((BB, 1), lambda i: (i, 0)),
            pl.BlockSpec((N_CHANS, FEAT), lambda i: (0, 0)),
            pl.BlockSpec((1, FEAT), lambda i: (0, 0)),
            pl.BlockSpec((FEAT, E * N_OUT), lambda i: (0, 0)),
            pl.BlockSpec((1, E * N_OUT), lambda i: (0, 0)),
        ],
        out_specs=pl.BlockSpec((BB, N_OUT), lambda i: (i, 0)),
        out_shape=jax.ShapeDtypeStruct((B, N_OUT), jnp.float32),
    )(x, sid2, W_bb, bbb, W_all, b_all)


# speedup vs baseline: 4.0731x; 4.0731x over previous
import jax
import jax.numpy as jnp
from jax.experimental import pallas as pl
from jax.experimental.pallas import tpu as pltpu

B = 1024
N_CHANS = 64
WINDOW = 1000
N_OUT = 4
E = 16
FEAT = 512

WB = 40
NSTEP = WINDOW // WB


def _tc_body(xT_ref, sid_ref, Wbb_ref, bbb_ref, Wall_ref, ball_ref,
             outT_ref, acc_ref):
    i = pl.program_id(0)

    @pl.when(i == 0)
    def _():
        acc_ref[...] = jnp.zeros_like(acc_ref)

    acc_ref[...] += jnp.sum(xT_ref[...], axis=1)

    @pl.when(i == NSTEP - 1)
    def _():
        m = acc_ref[...] * (1.0 / WINDOW)
        dn = (((0,), (0,)), ((), ()))
        featsT = jax.lax.dot_general(Wbb_ref[...], m, dn,
                                     preferred_element_type=jnp.float32)
        featsT = jnp.maximum(featsT + bbb_ref[...], 0.0)
        allhT = jax.lax.dot_general(Wall_ref[...], featsT, dn,
                                    preferred_element_type=jnp.float32)
        allhT = allhT + ball_ref[...]
        sid = sid_ref[...]
        row = jax.lax.broadcasted_iota(jnp.int32, (E * N_OUT, B), 0)
        mask = (row // N_OUT == sid).astype(jnp.float32)
        jo = jax.lax.broadcasted_iota(jnp.int32, (E * N_OUT, N_OUT), 0)
        oo = jax.lax.broadcasted_iota(jnp.int32, (E * N_OUT, N_OUT), 1)
        sel = (jo % N_OUT == oo).astype(jnp.float32)
        outT_ref[...] = jax.lax.dot_general(sel, allhT * mask, dn,
                                            preferred_element_type=jnp.float32)


@jax.jit
def kernel(x, subject_ids, W_bb, b_bb, W_heads, b_heads):
    xT = jnp.transpose(x, (1, 2, 0))
    sid = subject_ids.astype(jnp.int32).reshape(1, B)
    W_all = W_heads.transpose(1, 0, 2).reshape(FEAT, E * N_OUT)
    b_all = b_heads.reshape(E * N_OUT, 1)
    bbb = b_bb.reshape(FEAT, 1)
    outT = pl.pallas_call(
        _tc_body,
        grid=(NSTEP,),
        in_specs=[
            pl.BlockSpec((N_CHANS, WB, B), lambda i: (0, i, 0)),
            pl.BlockSpec((1, B), lambda i: (0, 0)),
            pl.BlockSpec((N_CHANS, FEAT), lambda i: (0, 0)),
            pl.BlockSpec((FEAT, 1), lambda i: (0, 0)),
            pl.BlockSpec((FEAT, E * N_OUT), lambda i: (0, 0)),
            pl.BlockSpec((E * N_OUT, 1), lambda i: (0, 0)),
        ],
        out_specs=pl.BlockSpec((N_OUT, B), lambda i: (0, 0)),
        out_shape=jax.ShapeDtypeStruct((N_OUT, B), jnp.float32),
        scratch_shapes=[pltpu.VMEM((N_CHANS, B), jnp.float32)],
    )(xT, sid, W_bb, bbb, W_all, b_all)
    return outT.T
